# baseline (speedup 1.0000x reference)
import functools

import jax
import jax.numpy as jnp
from jax import lax
from jax.experimental import pallas as pl
from jax.experimental.pallas import tpu as pltpu
from jax.experimental.pallas import tpu_sc as plsc

_TAU = 0.1
_EPS = 1e-8
_L = 16
_TR = 8
_TC = 128


def _topk_body(x_ref, w1_ref, b1_ref, w2_ref, b2_ref, g_ref, idx_ref, w_ref):
    x = x_ref[...]
    h = jnp.maximum(
        lax.dot_general(x, w1_ref[...], (((1,), (0,)), ((), ())),
                        preferred_element_type=jnp.float32)
        + b1_ref[...][None, :], 0.0)
    q = lax.dot_general(h, w2_ref[...], (((1,), (0,)), ((), ())),
                        preferred_element_type=jnp.float32) + b2_ref[...][None, :]
    qn = q / jnp.maximum(jnp.sqrt(jnp.sum(q * q, axis=1, keepdims=True)), _EPS)
    g = g_ref[...]
    gn = g / jnp.maximum(jnp.sqrt(jnp.sum(g * g, axis=1, keepdims=True)), _EPS)
    sim = lax.dot_general(qn, gn, (((1,), (1,)), ((), ())),
                          preferred_element_type=jnp.float32)
    n = sim.shape[1]
    ii = lax.broadcasted_iota(jnp.int32, sim.shape, 1)
    m1 = jnp.max(sim, axis=1, keepdims=True)
    i1 = jnp.min(jnp.where(sim == m1, ii, n), axis=1, keepdims=True)
    sim2 = jnp.where(ii == i1, -jnp.inf, sim)
    m2 = jnp.max(sim2, axis=1, keepdims=True)
    i2 = jnp.min(jnp.where(sim2 == m2, ii, n), axis=1, keepdims=True)
    e = jnp.exp((m2 - m1) / _TAU)
    denom = 1.0 + e
    idx_ref[...] = jnp.concatenate([i1, i2], axis=1)
    w_ref[...] = jnp.concatenate([1.0 / denom, e / denom], axis=1)


def _topk_tc(query_encoding, W1, b1, W2, b2, gallery_embeddings):
    B = query_encoding.shape[0]
    return pl.pallas_call(
        _topk_body,
        out_shape=(
            jax.ShapeDtypeStruct((B, 2), jnp.int32),
            jax.ShapeDtypeStruct((B, 2), jnp.float32),
        ),
    )(query_encoding, W1, b1, W2, b2, gallery_embeddings)


def _gather_combine_sc(imgs_flat, masks_flat, ai_flat, am_flat, wb, dh, mh):
    nrow = wb.shape[0]
    mesh = plsc.VectorSubcoreMesh(core_axis_name="c", subcore_axis_name="s")

    @functools.partial(
        pl.kernel,
        out_type=(
            jax.ShapeDtypeStruct((nrow, dh), jnp.float32),
            jax.ShapeDtypeStruct((nrow, mh), jnp.float32),
        ),
        mesh=mesh,
        scratch_types=[
            pltpu.VMEM((2, _L), jnp.float32),
            pltpu.VMEM((dh,), jnp.int32),
            pltpu.VMEM((dh,), jnp.int32),
            pltpu.VMEM((dh,), jnp.float32),
            pltpu.VMEM((dh,), jnp.float32),
            pltpu.VMEM((mh,), jnp.int32),
            pltpu.VMEM((mh,), jnp.int32),
            pltpu.VMEM((mh,), jnp.float32),
            pltpu.VMEM((mh,), jnp.float32),
            pltpu.VMEM((dh,), jnp.float32),
            pltpu.VMEM((mh,), jnp.float32),
            pltpu.SemaphoreType.DMA,
            pltpu.SemaphoreType.DMA,
            pltpu.SemaphoreType.DMA,
        ],
    )
    def k(imgs_hbm, masks_hbm, ai_hbm, am_hbm, wb_hbm, oi_hbm, om_hbm,
          w_v, ii0, ii1, ci0, ci1, im0, im1, cm0, cm1,
          acc_i, acc_m, sem1, sem2, sem3):
        wid = lax.axis_index("s") * 2 + lax.axis_index("c")
        cpw = pltpu.async_copy(wb_hbm.at[wid], w_v, sem3)
        pre = [
            pltpu.async_copy(ai_hbm.at[wid, 0], ii0, sem3),
            pltpu.async_copy(am_hbm.at[wid, 0], im0, sem3),
            pltpu.async_copy(ai_hbm.at[wid, 1], ii1, sem3),
            pltpu.async_copy(am_hbm.at[wid, 1], im1, sem3),
        ]
        cpw.wait()
        cps = []
        for cp, (ib, cb, tab, sem) in zip(pre, (
                (ii0, ci0, imgs_hbm, sem1), (im0, cm0, masks_hbm, sem2),
                (ii1, ci1, imgs_hbm, sem1), (im1, cm1, masks_hbm, sem2))):
            cp.wait()
            cps.append(pltpu.async_copy(tab.at[ib], cb, sem))

        w0 = w_v[0]
        w1 = w_v[1]
        cps[0].wait()
        cps[2].wait()

        def ci(j, carry):
            s = pl.ds(j * _L, _L)
            acc_i[s] = w0 * ci0[s] + w1 * ci1[s]
            return carry

        lax.fori_loop(0, dh // _L, ci, 0)
        pltpu.sync_copy(acc_i, oi_hbm.at[wid])
        cps[1].wait()
        cps[3].wait()

        def cm(j, carry):
            s = pl.ds(j * _L, _L)
            acc_m[s] = w0 * cm0[s] + w1 * cm1[s]
            return carry

        lax.fori_loop(0, mh // _L, cm, 0)
        pltpu.sync_copy(acc_m, om_hbm.at[wid])

    return k(imgs_flat, masks_flat, ai_flat, am_flat, wb)


def kernel(query_encoding, W1, b1, W2, b2, gallery_embeddings,
           gallery_images, gallery_masks):
    B = query_encoding.shape[0]
    N, C, H, W = gallery_images.shape
    D = C * H * W
    M = H * W
    R = 2 * B

    topk_idx, topk_w = _topk_tc(query_encoding, W1, b1, W2, b2,
                                gallery_embeddings)

    imgs_flat = (gallery_images.transpose(1, 2, 3, 0)
                 .reshape(C, H, W // _TR, _TR, N // _TC, _TC)
                 .transpose(0, 1, 2, 4, 3, 5).reshape(-1))
    masks_flat = (gallery_masks.transpose(1, 2, 0)
                  .reshape(H, W // _TR, _TR, N // _TC, _TC)
                  .transpose(0, 1, 3, 2, 4).reshape(-1))
    dh, mh = D // 2, M // 2
    rep_idx = jnp.repeat(topk_idx, 2, axis=0)
    wb = jnp.broadcast_to(
        jnp.repeat(topk_w, 2, axis=0)[:, :, None], (R, 2, _L))
    half = (jnp.arange(R, dtype=jnp.int32) % 2)
    key = (rep_idx // _TC) * 1024 + rep_idx % _TC
    qi = half[:, None, None] * dh + jnp.arange(dh, dtype=jnp.int32)
    ai = (qi // _TR) * 32768 + (qi % _TR) * _TC + key[:, :, None]
    qm = half[:, None, None] * mh + jnp.arange(mh, dtype=jnp.int32)
    am = (qm // _TR) * 32768 + (qm % _TR) * _TC + key[:, :, None]

    gi, gm = _gather_combine_sc(imgs_flat, masks_flat, ai, am, wb, dh, mh)

    guide_image = gi.reshape(B, C, H, W)
    guide_mask = gm.reshape(B, H, W)
    return guide_image, guide_mask


# speedup vs baseline: 1.0314x; 1.0314x over previous
import functools

import jax
import jax.numpy as jnp
from jax import lax
from jax.experimental import pallas as pl
from jax.experimental.pallas import tpu as pltpu
from jax.experimental.pallas import tpu_sc as plsc

_TAU = 0.1
_EPS = 1e-8
_L = 16
_TR = 8
_TC = 128


def _topk_body(x_ref, w1_ref, b1_ref, w2_ref, b2_ref, g_ref, idx_ref, w_ref):
    x = x_ref[...]
    h = jnp.maximum(
        lax.dot_general(x, w1_ref[...], (((1,), (0,)), ((), ())),
                        preferred_element_type=jnp.float32)
        + b1_ref[...][None, :], 0.0)
    q = lax.dot_general(h, w2_ref[...], (((1,), (0,)), ((), ())),
                        preferred_element_type=jnp.float32) + b2_ref[...][None, :]
    qn = q / jnp.maximum(jnp.sqrt(jnp.sum(q * q, axis=1, keepdims=True)), _EPS)
    g = g_ref[...]
    gn = g / jnp.maximum(jnp.sqrt(jnp.sum(g * g, axis=1, keepdims=True)), _EPS)
    sim = lax.dot_general(qn, gn, (((1,), (1,)), ((), ())),
                          preferred_element_type=jnp.float32)
    n = sim.shape[1]
    ii = lax.broadcasted_iota(jnp.int32, sim.shape, 1)
    m1 = jnp.max(sim, axis=1, keepdims=True)
    i1 = jnp.min(jnp.where(sim == m1, ii, n), axis=1, keepdims=True)
    sim2 = jnp.where(ii == i1, -jnp.inf, sim)
    m2 = jnp.max(sim2, axis=1, keepdims=True)
    i2 = jnp.min(jnp.where(sim2 == m2, ii, n), axis=1, keepdims=True)
    e = jnp.exp((m2 - m1) / _TAU)
    denom = 1.0 + e
    idx_ref[...] = jnp.concatenate([i1, i2], axis=1)
    w_ref[...] = jnp.concatenate([1.0 / denom, e / denom], axis=1)


def _topk_tc(query_encoding, W1, b1, W2, b2, gallery_embeddings):
    B = query_encoding.shape[0]
    return pl.pallas_call(
        _topk_body,
        out_shape=(
            jax.ShapeDtypeStruct((B, 2), jnp.int32),
            jax.ShapeDtypeStruct((B, 2), jnp.float32),
        ),
    )(query_encoding, W1, b1, W2, b2, gallery_embeddings)


def _gather_combine_sc(imgs_flat, masks_flat, ai_flat, am_flat, wb, dh, mh):
    nrow = wb.shape[0]
    mesh = plsc.VectorSubcoreMesh(core_axis_name="c", subcore_axis_name="s")

    hh = dh // 2

    @functools.partial(
        pl.kernel,
        out_type=(
            jax.ShapeDtypeStruct((nrow, dh), jnp.float32),
            jax.ShapeDtypeStruct((nrow, mh), jnp.float32),
        ),
        mesh=mesh,
        scratch_types=[
            pltpu.VMEM((2, _L), jnp.float32),
            pltpu.VMEM((hh,), jnp.int32),
            pltpu.VMEM((hh,), jnp.int32),
            pltpu.VMEM((hh,), jnp.int32),
            pltpu.VMEM((hh,), jnp.int32),
            pltpu.VMEM((dh,), jnp.float32),
            pltpu.VMEM((dh,), jnp.float32),
            pltpu.VMEM((mh,), jnp.int32),
            pltpu.VMEM((mh,), jnp.int32),
            pltpu.VMEM((mh,), jnp.float32),
            pltpu.VMEM((mh,), jnp.float32),
            pltpu.VMEM((dh,), jnp.float32),
            pltpu.VMEM((mh,), jnp.float32),
            pltpu.SemaphoreType.DMA,
            pltpu.SemaphoreType.DMA,
            pltpu.SemaphoreType.DMA,
            pltpu.SemaphoreType.DMA,
        ],
    )
    def k(imgs_hbm, masks_hbm, ai_hbm, am_hbm, wb_hbm, oi_hbm, om_hbm,
          w_v, i0a, i0b, i1a, i1b, ci0, ci1, im0, im1, cm0, cm1,
          acc_i, acc_m, semA, semB, semC, semP):
        wid = lax.axis_index("s") * 2 + lax.axis_index("c")
        cpw = pltpu.async_copy(wb_hbm.at[wid], w_v, semP)
        pre = [
            pltpu.async_copy(ai_hbm.at[wid, 0, pl.ds(0, hh)], i0a, semP),
            pltpu.async_copy(ai_hbm.at[wid, 0, pl.ds(hh, hh)], i0b, semP),
            pltpu.async_copy(ai_hbm.at[wid, 1, pl.ds(0, hh)], i1a, semP),
            pltpu.async_copy(ai_hbm.at[wid, 1, pl.ds(hh, hh)], i1b, semP),
            pltpu.async_copy(am_hbm.at[wid, 0], im0, semP),
            pltpu.async_copy(am_hbm.at[wid, 1], im1, semP),
        ]
        cpw.wait()
        specs = (
            (i0a, ci0.at[pl.ds(0, hh)], imgs_hbm, semA),
            (i0b, ci0.at[pl.ds(hh, hh)], imgs_hbm, semB),
            (i1a, ci1.at[pl.ds(0, hh)], imgs_hbm, semA),
            (i1b, ci1.at[pl.ds(hh, hh)], imgs_hbm, semB),
            (im0, cm0, masks_hbm, semC),
            (im1, cm1, masks_hbm, semC),
        )
        cps = []
        for cp, (ib, cb, tab, sem) in zip(pre, specs):
            cp.wait()
            cps.append(pltpu.async_copy(tab.at[ib], cb, sem))

        w0 = w_v[0]
        w1 = w_v[1]
        for cp in cps[:4]:
            cp.wait()

        def ci(j, carry):
            s0 = pl.ds(j * 2 * _L, _L)
            s1 = pl.ds(j * 2 * _L + _L, _L)
            acc_i[s0] = w0 * ci0[s0] + w1 * ci1[s0]
            acc_i[s1] = w0 * ci0[s1] + w1 * ci1[s1]
            return carry

        lax.fori_loop(0, dh // (2 * _L), ci, 0)
        pltpu.sync_copy(acc_i, oi_hbm.at[wid])
        cps[4].wait()
        cps[5].wait()

        def cm(j, carry):
            s0 = pl.ds(j * 2 * _L, _L)
            s1 = pl.ds(j * 2 * _L + _L, _L)
            acc_m[s0] = w0 * cm0[s0] + w1 * cm1[s0]
            acc_m[s1] = w0 * cm0[s1] + w1 * cm1[s1]
            return carry

        lax.fori_loop(0, mh // (2 * _L), cm, 0)
        pltpu.sync_copy(acc_m, om_hbm.at[wid])

    return k(imgs_flat, masks_flat, ai_flat, am_flat, wb)


def kernel(query_encoding, W1, b1, W2, b2, gallery_embeddings,
           gallery_images, gallery_masks):
    B = query_encoding.shape[0]
    N, C, H, W = gallery_images.shape
    D = C * H * W
    M = H * W
    R = 2 * B

    topk_idx, topk_w = _topk_tc(query_encoding, W1, b1, W2, b2,
                                gallery_embeddings)

    imgs_flat = (gallery_images.transpose(1, 2, 3, 0)
                 .reshape(C, H, W // _TR, _TR, N // _TC, _TC)
                 .transpose(0, 1, 2, 4, 3, 5).reshape(-1))
    masks_flat = (gallery_masks.transpose(1, 2, 0)
                  .reshape(H, W // _TR, _TR, N // _TC, _TC)
                  .transpose(0, 1, 3, 2, 4).reshape(-1))
    dh, mh = D // 2, M // 2
    rep_idx = jnp.repeat(topk_idx, 2, axis=0)
    wb = jnp.broadcast_to(
        jnp.repeat(topk_w, 2, axis=0)[:, :, None], (R, 2, _L))
    half = (jnp.arange(R, dtype=jnp.int32) % 2)
    key = (rep_idx // _TC) * 1024 + rep_idx % _TC
    qi = half[:, None, None] * dh + jnp.arange(dh, dtype=jnp.int32)
    ai = (qi // _TR) * 32768 + (qi % _TR) * _TC + key[:, :, None]
    qm = half[:, None, None] * mh + jnp.arange(mh, dtype=jnp.int32)
    am = (qm // _TR) * 32768 + (qm % _TR) * _TC + key[:, :, None]

    gi, gm = _gather_combine_sc(imgs_flat, masks_flat, ai, am, wb, dh, mh)

    guide_image = gi.reshape(B, C, H, W)
    guide_mask = gm.reshape(B, H, W)
    return guide_image, guide_mask


# speedup vs baseline: 1.0345x; 1.0030x over previous
import functools

import jax
import jax.numpy as jnp
from jax import lax
from jax.experimental import pallas as pl
from jax.experimental.pallas import tpu as pltpu
from jax.experimental.pallas import tpu_sc as plsc

_TAU = 0.1
_EPS = 1e-8
_L = 16
_TR = 8
_TC = 128


def _topk_body(x_ref, w1_ref, b1_ref, w2_ref, b2_ref, g_ref, idx_ref, w_ref):
    x = x_ref[...]
    h = jnp.maximum(
        lax.dot_general(x, w1_ref[...], (((1,), (0,)), ((), ())),
                        preferred_element_type=jnp.float32)
        + b1_ref[...][None, :], 0.0)
    q = lax.dot_general(h, w2_ref[...], (((1,), (0,)), ((), ())),
                        preferred_element_type=jnp.float32) + b2_ref[...][None, :]
    qn = q / jnp.maximum(jnp.sqrt(jnp.sum(q * q, axis=1, keepdims=True)), _EPS)
    g = g_ref[...]
    gn = g / jnp.maximum(jnp.sqrt(jnp.sum(g * g, axis=1, keepdims=True)), _EPS)
    sim = lax.dot_general(qn, gn, (((1,), (1,)), ((), ())),
                          preferred_element_type=jnp.float32)
    n = sim.shape[1]
    ii = lax.broadcasted_iota(jnp.int32, sim.shape, 1)
    m1 = jnp.max(sim, axis=1, keepdims=True)
    i1 = jnp.min(jnp.where(sim == m1, ii, n), axis=1, keepdims=True)
    sim2 = jnp.where(ii == i1, -jnp.inf, sim)
    m2 = jnp.max(sim2, axis=1, keepdims=True)
    i2 = jnp.min(jnp.where(sim2 == m2, ii, n), axis=1, keepdims=True)
    e = jnp.exp((m2 - m1) / _TAU)
    denom = 1.0 + e
    idx_ref[...] = jnp.concatenate([i1, i2], axis=1)
    w_ref[...] = jnp.concatenate([1.0 / denom, e / denom], axis=1)


def _topk_tc(query_encoding, W1, b1, W2, b2, gallery_embeddings):
    B = query_encoding.shape[0]
    return pl.pallas_call(
        _topk_body,
        out_shape=(
            jax.ShapeDtypeStruct((B, 2), jnp.int32),
            jax.ShapeDtypeStruct((B, 2), jnp.float32),
        ),
    )(query_encoding, W1, b1, W2, b2, gallery_embeddings)


def _gather_combine_sc(imgs_flat, masks_flat, ai_flat, am_flat, wb, dh, mh):
    nrow = wb.shape[0]
    mesh = plsc.VectorSubcoreMesh(core_axis_name="c", subcore_axis_name="s")

    hh = dh // 2

    @functools.partial(
        pl.kernel,
        out_type=(
            jax.ShapeDtypeStruct((nrow, dh), jnp.float32),
            jax.ShapeDtypeStruct((nrow, mh), jnp.float32),
        ),
        mesh=mesh,
        scratch_types=[
            pltpu.VMEM((2, _L), jnp.float32),
            pltpu.VMEM((hh,), jnp.int32),
            pltpu.VMEM((hh,), jnp.int32),
            pltpu.VMEM((hh,), jnp.int32),
            pltpu.VMEM((hh,), jnp.int32),
            pltpu.VMEM((dh,), jnp.float32),
            pltpu.VMEM((dh,), jnp.float32),
            pltpu.VMEM((mh,), jnp.int32),
            pltpu.VMEM((mh,), jnp.int32),
            pltpu.VMEM((mh,), jnp.float32),
            pltpu.VMEM((mh,), jnp.float32),
            pltpu.VMEM((dh,), jnp.float32),
            pltpu.VMEM((mh,), jnp.float32),
            pltpu.SemaphoreType.DMA,
            pltpu.SemaphoreType.DMA,
            pltpu.SemaphoreType.DMA,
            pltpu.SemaphoreType.DMA,
        ],
    )
    def k(imgs_hbm, masks_hbm, ai_hbm, am_hbm, wb_hbm, oi_hbm, om_hbm,
          w_v, i0a, i0b, i1a, i1b, ci0, ci1, im0, im1, cm0, cm1,
          acc_i, acc_m, semA, semB, semC, semP):
        wid = lax.axis_index("s") * 2 + lax.axis_index("c")
        cpw = pltpu.async_copy(wb_hbm.at[wid], w_v, semP)
        pre = [
            pltpu.async_copy(ai_hbm.at[wid, 0, pl.ds(0, hh)], i0a, semP),
            pltpu.async_copy(ai_hbm.at[wid, 1, pl.ds(0, hh)], i1a, semP),
            pltpu.async_copy(ai_hbm.at[wid, 0, pl.ds(hh, hh)], i0b, semP),
            pltpu.async_copy(ai_hbm.at[wid, 1, pl.ds(hh, hh)], i1b, semP),
            pltpu.async_copy(am_hbm.at[wid, 0], im0, semP),
            pltpu.async_copy(am_hbm.at[wid, 1], im1, semP),
        ]
        specs = (
            (i0a, ci0.at[pl.ds(0, hh)], imgs_hbm, semA),
            (i1a, ci1.at[pl.ds(0, hh)], imgs_hbm, semA),
            (i0b, ci0.at[pl.ds(hh, hh)], imgs_hbm, semB),
            (i1b, ci1.at[pl.ds(hh, hh)], imgs_hbm, semB),
            (im0, cm0, masks_hbm, semC),
            (im1, cm1, masks_hbm, semC),
        )
        cps = []
        for cp, (ib, cb, tab, sem) in zip(pre, specs):
            cp.wait()
            cps.append(pltpu.async_copy(tab.at[ib], cb, sem))
        cpw.wait()

        w0 = w_v[0]
        w1 = w_v[1]
        cps[0].wait()
        cps[1].wait()

        def ci(j, carry):
            s0 = pl.ds(j * 2 * _L, _L)
            s1 = pl.ds(j * 2 * _L + _L, _L)
            acc_i[s0] = w0 * ci0[s0] + w1 * ci1[s0]
            acc_i[s1] = w0 * ci0[s1] + w1 * ci1[s1]
            return carry

        lax.fori_loop(0, hh // (2 * _L), ci, 0)
        cps[2].wait()
        cps[3].wait()
        lax.fori_loop(hh // (2 * _L), dh // (2 * _L), ci, 0)
        pltpu.sync_copy(acc_i, oi_hbm.at[wid])
        cps[4].wait()
        cps[5].wait()

        def cm(j, carry):
            s0 = pl.ds(j * 2 * _L, _L)
            s1 = pl.ds(j * 2 * _L + _L, _L)
            acc_m[s0] = w0 * cm0[s0] + w1 * cm1[s0]
            acc_m[s1] = w0 * cm0[s1] + w1 * cm1[s1]
            return carry

        lax.fori_loop(0, mh // (2 * _L), cm, 0)
        pltpu.sync_copy(acc_m, om_hbm.at[wid])

    return k(imgs_flat, masks_flat, ai_flat, am_flat, wb)


def kernel(query_encoding, W1, b1, W2, b2, gallery_embeddings,
           gallery_images, gallery_masks):
    B = query_encoding.shape[0]
    N, C, H, W = gallery_images.shape
    D = C * H * W
    M = H * W
    R = 2 * B

    topk_idx, topk_w = _topk_tc(query_encoding, W1, b1, W2, b2,
                                gallery_embeddings)

    imgs_flat = (gallery_images.transpose(1, 2, 3, 0)
                 .reshape(C, H, W // _TR, _TR, N // _TC, _TC)
                 .transpose(0, 1, 2, 4, 3, 5).reshape(-1))
    masks_flat = (gallery_masks.transpose(1, 2, 0)
                  .reshape(H, W // _TR, _TR, N // _TC, _TC)
                  .transpose(0, 1, 3, 2, 4).reshape(-1))
    dh, mh = D // 2, M // 2
    rep_idx = jnp.repeat(topk_idx, 2, axis=0)
    wb = jnp.broadcast_to(
        jnp.repeat(topk_w, 2, axis=0)[:, :, None], (R, 2, _L))
    half = (jnp.arange(R, dtype=jnp.int32) % 2)
    key = (rep_idx // _TC) * 1024 + rep_idx % _TC
    qi = half[:, None, None] * dh + jnp.arange(dh, dtype=jnp.int32)
    ai = (qi // _TR) * 32768 + (qi % _TR) * _TC + key[:, :, None]
    qm = half[:, None, None] * mh + jnp.arange(mh, dtype=jnp.int32)
    am = (qm // _TR) * 32768 + (qm % _TR) * _TC + key[:, :, None]

    gi, gm = _gather_combine_sc(imgs_flat, masks_flat, ai, am, wb, dh, mh)

    guide_image = gi.reshape(B, C, H, W)
    guide_mask = gm.reshape(B, H, W)
    return guide_image, guide_mask
